# Initial kernel scaffold; baseline (speedup 1.0000x reference)
import jax
import jax.numpy as jnp
from jax.experimental import pallas as pl


def kernel(a_indices, b_indices, coefficients):
    raise NotImplementedError("write your pallas kernel here")


# speedup vs baseline: 2.4320x; 2.4320x over previous
import functools

import jax
import jax.numpy as jnp
from jax import lax
from jax.experimental import pallas as pl
from jax.experimental.pallas import tpu as pltpu
from jax.experimental.pallas import tpu_sc as plsc

NC, NS, L = 2, 16, 16
NW = NC * NS
NUM_IDX = 16384
D = 128
PER_W = NUM_IDX // NW
CH = 128
NCHUNK = PER_W // CH
G = D // L

_mesh = plsc.VectorSubcoreMesh(
    core_axis_name="c", subcore_axis_name="s", num_cores=NC, num_subcores=NS)


@functools.partial(
    pl.kernel,
    out_type=jax.ShapeDtypeStruct((NC, L), jnp.float32),
    mesh=_mesh,
    compiler_params=pltpu.CompilerParams(needs_layout_passes=False),
    scratch_types=[
        pltpu.VMEM((PER_W,), jnp.int32),
        pltpu.VMEM((CH, D), jnp.float32),
        pltpu.VMEM((CH, D), jnp.float32),
        pltpu.VMEM((NS, D), jnp.float32),
        pltpu.VMEM((D,), jnp.int32),
        pltpu.VMEM((D,), jnp.float32),
        pltpu.VMEM((L,), jnp.float32),
        pltpu.VMEM_SHARED((NS, D), jnp.float32),
        pltpu.SemaphoreType.DMA,
        pltpu.SemaphoreType.DMA,
    ],
)
def _sum_kernel(a_hbm, b_hbm, coef_hbm, out_hbm,
                idx_v, buf0, buf1, gath_v, bidx_v, colsum_v, out_v,
                shared, sem0, sem1):
    cid = lax.axis_index("c")
    sid = lax.axis_index("s")
    wid = cid * NS + sid
    base = wid * PER_W

    pltpu.sync_copy(a_hbm.at[pl.ds(base, PER_W)], idx_v)

    bufs = (buf0, buf1)
    sems = (sem0, sem1)

    def start(c):
        return pltpu.async_copy(
            coef_hbm.at[idx_v.at[pl.ds(c * CH, CH)]], bufs[c % 2], sems[c % 2])

    def accum(buf, accs):
        def body(r, a):
            return tuple(a[g] + buf[r, pl.ds(g * L, L)] for g in range(G))
        return lax.fori_loop(0, CH, body, accs)

    accs = tuple(jnp.zeros((L,), jnp.float32) for _ in range(G))
    handles = [None] * NCHUNK
    handles[0] = start(0)
    for c in range(NCHUNK):
        if c + 1 < NCHUNK:
            handles[c + 1] = start(c + 1)
        handles[c].wait()
        accs = accum(bufs[c % 2], accs)

    for g in range(G):
        colsum_v[pl.ds(g * L, L)] = accs[g]
    pltpu.sync_copy(colsum_v, shared.at[sid])
    plsc.subcore_barrier()

    @pl.when(sid == 0)
    def _():
        pltpu.sync_copy(shared, gath_v)
        pltpu.sync_copy(b_hbm, bidx_v)
        for g in range(G):
            acc = gath_v[0, pl.ds(g * L, L)]
            for s in range(1, NS):
                acc = acc + gath_v[s, pl.ds(g * L, L)]
            colsum_v[pl.ds(g * L, L)] = acc
        tot = jnp.zeros((L,), jnp.float32)
        for g in range(G):
            idxg = bidx_v[pl.ds(g * L, L)]
            tot = tot + plsc.load_gather(colsum_v, [idxg])
        s_val = jnp.sum(tot)
        lane = lax.iota(jnp.int32, L)
        out_v[...] = jnp.where(lane == 0, s_val, jnp.float32(0.0))
        pltpu.sync_copy(out_v, out_hbm.at[cid])


def kernel(a_indices, b_indices, coefficients):
    out = _sum_kernel(a_indices.astype(jnp.int32),
                      b_indices.astype(jnp.int32),
                      coefficients)
    return jnp.sum(out)


# speedup vs baseline: 2.5029x; 1.0291x over previous
import functools

import jax
import jax.numpy as jnp
from jax import lax
from jax.experimental import pallas as pl
from jax.experimental.pallas import tpu as pltpu
from jax.experimental.pallas import tpu_sc as plsc

NC, NS, L = 2, 16, 16
NW = NC * NS
NUM_IDX = 16384
D = 128
PER_W = NUM_IDX // NW
CH = 128
NCHUNK = PER_W // CH
G = D // L

_mesh = plsc.VectorSubcoreMesh(
    core_axis_name="c", subcore_axis_name="s", num_cores=NC, num_subcores=NS)


@functools.partial(
    pl.kernel,
    out_type=jax.ShapeDtypeStruct((NC, L), jnp.float32),
    mesh=_mesh,
    compiler_params=pltpu.CompilerParams(
        needs_layout_passes=False,
        disable_bounds_checks=True,
        disable_semaphore_checks=True,
    ),
    scratch_types=[
        pltpu.VMEM((PER_W,), jnp.int32),
        pltpu.VMEM((NCHUNK, CH, D), jnp.float32),
        pltpu.VMEM((NS, D), jnp.float32),
        pltpu.VMEM((D,), jnp.int32),
        pltpu.VMEM((D,), jnp.float32),
        pltpu.VMEM((L,), jnp.float32),
        pltpu.VMEM_SHARED((NS, D), jnp.float32),
        pltpu.SemaphoreType.DMA,
    ],
)
def _sum_kernel(a_hbm, b_hbm, coef_hbm, out_hbm,
                idx_v, bufs, gath_v, bidx_v, colsum_v, out_v,
                shared, sem):
    cid = lax.axis_index("c")
    sid = lax.axis_index("s")
    wid = cid * NS + sid
    base = wid * PER_W

    pltpu.sync_copy(a_hbm.at[pl.ds(base, PER_W)], idx_v)

    handles = [
        pltpu.async_copy(
            coef_hbm.at[idx_v.at[pl.ds(c * CH, CH)]], bufs.at[c], sem)
        for c in range(NCHUNK)
    ]

    ROW_U = 4

    def accum(c, accs):
        buf = bufs.at[c]
        def body(r0, a):
            a = list(a)
            for u in range(ROW_U):
                r = r0 * ROW_U + u
                for g in range(G):
                    a[g] = a[g] + buf[r, pl.ds(g * L, L)]
            return tuple(a)
        return lax.fori_loop(0, CH // ROW_U, body, accs)

    accs = tuple(jnp.zeros((L,), jnp.float32) for _ in range(G))
    for c in range(NCHUNK):
        handles[c].wait()
        accs = accum(c, accs)

    for g in range(G):
        colsum_v[pl.ds(g * L, L)] = accs[g]
    pltpu.sync_copy(colsum_v, shared.at[sid])
    plsc.subcore_barrier()

    @pl.when(sid == 0)
    def _():
        pltpu.sync_copy(shared, gath_v)
        pltpu.sync_copy(b_hbm, bidx_v)
        for g in range(G):
            acc = gath_v[0, pl.ds(g * L, L)]
            for s in range(1, NS):
                acc = acc + gath_v[s, pl.ds(g * L, L)]
            colsum_v[pl.ds(g * L, L)] = acc
        tot = jnp.zeros((L,), jnp.float32)
        for g in range(G):
            idxg = bidx_v[pl.ds(g * L, L)]
            tot = tot + plsc.load_gather(colsum_v, [idxg])
        s_val = jnp.sum(tot)
        lane = lax.iota(jnp.int32, L)
        out_v[...] = jnp.where(lane == 0, s_val, jnp.float32(0.0))
        pltpu.sync_copy(out_v, out_hbm.at[cid])


def kernel(a_indices, b_indices, coefficients):
    out = _sum_kernel(a_indices.astype(jnp.int32),
                      b_indices.astype(jnp.int32),
                      coefficients)
    return jnp.sum(out)


# speedup vs baseline: 2.5050x; 1.0008x over previous
import functools

import jax
import jax.numpy as jnp
from jax import lax
from jax.experimental import pallas as pl
from jax.experimental.pallas import tpu as pltpu
from jax.experimental.pallas import tpu_sc as plsc

NC, NS, L = 2, 16, 16
NW = NC * NS
NUM_IDX = 16384
D = 128
PER_W = NUM_IDX // NW
CH = 128
NCHUNK = PER_W // CH
G = D // L

_mesh = plsc.VectorSubcoreMesh(
    core_axis_name="c", subcore_axis_name="s", num_cores=NC, num_subcores=NS)


@functools.partial(
    pl.kernel,
    out_type=jax.ShapeDtypeStruct((NC, L), jnp.float32),
    mesh=_mesh,
    compiler_params=pltpu.CompilerParams(
        needs_layout_passes=False,
        disable_bounds_checks=True,
        disable_semaphore_checks=True,
        skip_device_barrier=True,
    ),
    scratch_types=[
        pltpu.VMEM((NCHUNK, CH), jnp.int32),
        pltpu.VMEM((NCHUNK, CH, D), jnp.float32),
        pltpu.VMEM((NS, D), jnp.float32),
        pltpu.VMEM((D,), jnp.int32),
        pltpu.VMEM((D,), jnp.float32),
        pltpu.VMEM((L,), jnp.float32),
        pltpu.VMEM_SHARED((NS, D), jnp.float32),
        pltpu.SemaphoreType.DMA,
        pltpu.SemaphoreType.DMA,
    ],
)
def _sum_kernel(a_hbm, b_hbm, coef_hbm, out_hbm,
                idx_v, bufs, gath_v, bidx_v, colsum_v, out_v,
                shared, isem, sem):
    cid = lax.axis_index("c")
    sid = lax.axis_index("s")
    wid = cid * NS + sid
    base = wid * PER_W

    ihandles = [
        pltpu.async_copy(a_hbm.at[pl.ds(base + c * CH, CH)], idx_v.at[c], isem)
        for c in range(NCHUNK)
    ]
    handles = []
    for c in range(NCHUNK):
        ihandles[c].wait()
        handles.append(
            pltpu.async_copy(coef_hbm.at[idx_v.at[c]], bufs.at[c], sem))

    ROW_U = 4

    def accum(c, accs):
        buf = bufs.at[c]
        def body(r0, a):
            a = list(a)
            for u in range(ROW_U):
                r = r0 * ROW_U + u
                for g in range(G):
                    a[g] = a[g] + buf[r, pl.ds(g * L, L)]
            return tuple(a)
        return lax.fori_loop(0, CH // ROW_U, body, accs)

    accs = tuple(jnp.zeros((L,), jnp.float32) for _ in range(G))
    for c in range(NCHUNK):
        handles[c].wait()
        accs = accum(c, accs)

    for g in range(G):
        colsum_v[pl.ds(g * L, L)] = accs[g]
    pltpu.sync_copy(colsum_v, shared.at[sid])
    plsc.subcore_barrier()

    @pl.when(sid == 0)
    def _():
        pltpu.sync_copy(shared, gath_v)
        pltpu.sync_copy(b_hbm, bidx_v)
        for g in range(G):
            acc = gath_v[0, pl.ds(g * L, L)]
            for s in range(1, NS):
                acc = acc + gath_v[s, pl.ds(g * L, L)]
            colsum_v[pl.ds(g * L, L)] = acc
        tot = jnp.zeros((L,), jnp.float32)
        for g in range(G):
            idxg = bidx_v[pl.ds(g * L, L)]
            tot = tot + plsc.load_gather(colsum_v, [idxg])
        s_val = jnp.sum(tot)
        lane = lax.iota(jnp.int32, L)
        out_v[...] = jnp.where(lane == 0, s_val, jnp.float32(0.0))
        pltpu.sync_copy(out_v, out_hbm.at[cid])


def kernel(a_indices, b_indices, coefficients):
    out = _sum_kernel(a_indices.astype(jnp.int32),
                      b_indices.astype(jnp.int32),
                      coefficients)
    return jnp.sum(out)
